# baseline (speedup 1.0000x reference)
import functools

import jax
import jax.numpy as jnp
from jax import lax
from jax.experimental import pallas as pl
from jax.experimental.pallas import tpu as pltpu
from jax.experimental.pallas import tpu_sc as plsc

NC = 2
NS = 16
NW = NC * NS
L = 16

EXP = 8
TOPK = 2
HID = 1024
INNER = 2048
ALPHA = 1.702
LIMIT = 7.0

T = 256
NBLK_MAX = 24
NMETA = 32


def _router_body(x_ref, rw_ref, rb_ref, idx_ref, w_ref):
    x = x_ref[...]
    logits = lax.dot_general(x, rw_ref[...], (((1,), (1,)), ((), ())),
                             preferred_element_type=jnp.float32)
    logits = logits + rb_ref[...][None, :]
    iota = lax.broadcasted_iota(jnp.int32, logits.shape, 1)
    m1 = jnp.max(logits, axis=-1, keepdims=True)
    i1 = jnp.min(jnp.where(logits == m1, iota, EXP), axis=-1, keepdims=True)
    neg = jnp.where(iota == i1, -jnp.inf, logits)
    m2 = jnp.max(neg, axis=-1, keepdims=True)
    i2 = jnp.min(jnp.where(neg == m2, iota, EXP), axis=-1, keepdims=True)
    w1 = 1.0 / (1.0 + jnp.exp(m2 - m1))
    idx_ref[...] = jnp.concatenate([i1, i2], axis=1)
    w_ref[...] = jnp.concatenate([w1, 1.0 - w1], axis=1)


def _gemm_body(be_ref, bv_ref, xs_ref, gu_ref, gub_ref, dp_ref, dpb_ref,
               ws_ref, out_ref):
    b = pl.program_id(0)

    @pl.when(bv_ref[b] == 1)
    def _():
        x = xs_ref[...].astype(jnp.bfloat16)
        h1 = lax.dot_general(x, gu_ref[0].astype(jnp.bfloat16),
                             (((1,), (1,)), ((), ())),
                             preferred_element_type=jnp.float32)
        h1 = h1 + gub_ref[0]
        gate = jnp.minimum(h1[:, :INNER], LIMIT)
        up = jnp.clip(h1[:, INNER:], -LIMIT, LIMIT)
        glu = gate * (1.0 / (1.0 + jnp.exp(-ALPHA * gate)))
        act = ((up + 1.0) * glu).astype(jnp.bfloat16)
        h2 = lax.dot_general(act, dp_ref[0].astype(jnp.bfloat16),
                             (((1,), (1,)), ((), ())),
                             preferred_element_type=jnp.float32)
        h2 = h2 + dpb_ref[0]
        out_ref[...] = h2 * ws_ref[0, 0][:, None]


def _make_dispatch(n_tok):
    n_assign = n_tok * TOPK
    padded = NBLK_MAX * T
    chunk = n_assign // NW
    tchunk = n_tok // NW
    mesh = plsc.VectorSubcoreMesh(core_axis_name="c", subcore_axis_name="s")

    @functools.partial(
        pl.kernel, mesh=mesh,
        compiler_params=pltpu.CompilerParams(needs_layout_passes=False),
        out_type=(jax.ShapeDtypeStruct((n_assign,), jnp.int32),
                  jax.ShapeDtypeStruct((padded, HID), jnp.float32),
                  jax.ShapeDtypeStruct((padded,), jnp.float32),
                  jax.ShapeDtypeStruct((NMETA,), jnp.int32),
                  jax.ShapeDtypeStruct((NMETA,), jnp.int32)),
        scratch_types=[
            pltpu.VMEM((n_assign,), jnp.int32),
            pltpu.VMEM((chunk,), jnp.float32),
            pltpu.VMEM((L,), jnp.int32),
            pltpu.VMEM((L,), jnp.int32),
            pltpu.VMEM((chunk,), jnp.int32),
            pltpu.VMEM((tchunk,), jnp.int32),
            pltpu.VMEM((tchunk,), jnp.int32),
            pltpu.VMEM((tchunk, HID), jnp.float32),
            pltpu.VMEM((2 * L,), jnp.int32),
            pltpu.VMEM((NMETA,), jnp.int32),
            pltpu.VMEM((NMETA,), jnp.int32),
            pltpu.SemaphoreType.DMA,
            pltpu.SemaphoreType.DMA,
        ])
    def dispatch(x_hbm, eid_hbm, wflat_hbm, pos_hbm, xs_hbm, wsort_hbm,
                 be_hbm, bv_hbm, eid_v, w_v, cnt_v, tab_v, slot_v, ev_v,
                 od_v, rows_v, end_v, meta_v, metb_v, sem, sem2):
        wid = lax.axis_index("s") * NC + lax.axis_index("c")
        base = wid * chunk
        rows_cp = pltpu.async_copy(
            x_hbm.at[pl.ds(wid * tchunk, tchunk)], rows_v, sem2)
        pltpu.sync_copy(eid_hbm, eid_v)
        pltpu.sync_copy(wflat_hbm.at[pl.ds(base, chunk)], w_v)
        ones = jnp.ones((L,), jnp.int32)
        cnt_v[...] = jnp.zeros((L,), jnp.int32)
        npre = wid * (chunk // L)

        def count_step(v, carry):
            vec = eid_v[pl.ds(v * L, L)]
            plsc.addupdate_scatter(cnt_v, [vec], ones)
            return carry

        lax.fori_loop(0, npre, count_step, 0, unroll=False)
        for j in range(chunk // L):
            vec = eid_v[pl.ds(base + j * L, L)]
            g = plsc.load_gather(cnt_v, [vec])
            wv = jnp.zeros((L,), jnp.int32)
            for e in range(EXP):
                m = vec == e
                c = jnp.cumsum(m.astype(jnp.int32))
                wv = jnp.where(m, c - 1, wv)
            slot_v[pl.ds(j * L, L)] = g + wv
            plsc.addupdate_scatter(cnt_v, [vec], ones)
        lax.fori_loop(npre + chunk // L, n_assign // L, count_step, 0,
                      unroll=False)
        total = cnt_v[...]
        nblk = (total + (T - 1)) // T
        end_incl = jnp.cumsum(nblk)
        tab_v[...] = (end_incl - nblk) * T
        lane = jnp.arange(L, dtype=jnp.int32)
        for j in range(chunk // L):
            vec = eid_v[pl.ds(base + j * L, L)]
            off = plsc.load_gather(tab_v, [vec])
            slot = slot_v[pl.ds(j * L, L)] + off
            slot_v[pl.ds(j * L, L)] = slot
            dst = j * (L // 2) + (lane >> 1)
            plsc.store_scatter(ev_v, [dst], slot, mask=(lane & 1) == 0)
            plsc.store_scatter(od_v, [dst], slot, mask=(lane & 1) == 1)
        pltpu.sync_copy(slot_v, pos_hbm.at[pl.ds(base, chunk)])
        rows_cp.wait()
        cp_e = pltpu.async_copy(rows_v, xs_hbm.at[ev_v], sem)
        cp_o = pltpu.async_copy(rows_v, xs_hbm.at[od_v], sem2)
        pltpu.async_copy(w_v, wsort_hbm.at[slot_v], sem).wait()
        cp_e.wait()
        cp_o.wait()

        @pl.when(wid == 0)
        def _():
            end_v[pl.ds(L, L)] = end_incl
            sel7 = jnp.full((L,), L + EXP - 1, jnp.int32)
            tot_blk = plsc.load_gather(end_v, [sel7])
            for c in range(NMETA // L):
                bvec = jnp.arange(L, dtype=jnp.int32) + c * L
                beff = jnp.minimum(bvec, tot_blk - 1)
                cv = jnp.zeros((L,), jnp.int32)
                for e in range(EXP):
                    end_e = plsc.load_gather(
                        end_v, [jnp.full((L,), L + e, jnp.int32)])
                    cv = cv + (beff >= end_e).astype(jnp.int32)
                meta_v[pl.ds(c * L, L)] = jnp.minimum(cv, EXP - 1)
                metb_v[pl.ds(c * L, L)] = (bvec < tot_blk).astype(jnp.int32)
            pltpu.sync_copy(meta_v, be_hbm)
            pltpu.sync_copy(metb_v, bv_hbm)

    return dispatch


def _make_combine(n_tok):
    padded = NBLK_MAX * T
    tchunk = n_tok // NW
    tc = 16
    mesh = plsc.VectorSubcoreMesh(core_axis_name="c", subcore_axis_name="s")

    @functools.partial(
        pl.kernel, mesh=mesh,
        out_type=jax.ShapeDtypeStruct((n_tok, HID), jnp.float32),
        scratch_types=[
            pltpu.VMEM((2 * tchunk,), jnp.int32),
            pltpu.VMEM((2 * tc, HID), jnp.float32),
            pltpu.VMEM((tc, HID), jnp.float32),
            pltpu.SemaphoreType.DMA,
        ])
    def combine(rows_hbm, pos_hbm, out_hbm, idx_v, pr_v, o_v, sem):
        wid = lax.axis_index("s") * NC + lax.axis_index("c")
        baset = wid * tchunk
        pltpu.sync_copy(pos_hbm.at[pl.ds(baset * TOPK, TOPK * tchunk)], idx_v)
        for c in range(tchunk // tc):
            pltpu.async_copy(
                rows_hbm.at[idx_v.at[pl.ds(c * TOPK * tc, TOPK * tc)]],
                pr_v, sem).wait()

            def tok_step(t, carry):
                for g in range(HID // L):
                    o_v[t, pl.ds(g * L, L)] = (
                        pr_v[2 * t, pl.ds(g * L, L)]
                        + pr_v[2 * t + 1, pl.ds(g * L, L)])
                return carry

            lax.fori_loop(0, tc, tok_step, 0, unroll=False)
            pltpu.sync_copy(o_v, out_hbm.at[pl.ds(baset + c * tc, tc)])

    return combine


def kernel(hidden_states, router_weight, router_bias, gate_up_proj,
           gate_up_proj_bias, down_proj, down_proj_bias):
    b, s, h = hidden_states.shape
    n_tok = b * s
    n_assign = n_tok * TOPK
    padded = NBLK_MAX * T
    x = hidden_states.reshape(n_tok, h)

    idx, w = pl.pallas_call(
        _router_body,
        out_shape=(jax.ShapeDtypeStruct((n_tok, TOPK), jnp.int32),
                   jax.ShapeDtypeStruct((n_tok, TOPK), jnp.float32)),
    )(x, router_weight, router_bias)

    eid = idx.reshape(n_assign)
    wflat = w.reshape(n_assign)

    pos, xs, wsort, be, bv = _make_dispatch(n_tok)(x, eid, wflat)

    ws3d = wsort.reshape(NBLK_MAX, 1, T)
    gub3d = gate_up_proj_bias.reshape(EXP, 1, 2 * INNER)
    dpb3d = down_proj_bias.reshape(EXP, 1, HID)
    grid_spec = pltpu.PrefetchScalarGridSpec(
        num_scalar_prefetch=2,
        grid=(NBLK_MAX,),
        in_specs=[
            pl.BlockSpec((T, HID), lambda i, be, bv: (i, 0)),
            pl.BlockSpec((1, 2 * INNER, HID), lambda i, be, bv: (be[i], 0, 0)),
            pl.BlockSpec((1, 1, 2 * INNER), lambda i, be, bv: (be[i], 0, 0)),
            pl.BlockSpec((1, HID, INNER), lambda i, be, bv: (be[i], 0, 0)),
            pl.BlockSpec((1, 1, HID), lambda i, be, bv: (be[i], 0, 0)),
            pl.BlockSpec((1, 1, T), lambda i, be, bv: (i, 0, 0)),
        ],
        out_specs=pl.BlockSpec((T, HID), lambda i, be, bv: (i, 0)),
    )
    rows = pl.pallas_call(
        _gemm_body,
        grid_spec=grid_spec,
        out_shape=jax.ShapeDtypeStruct((padded, HID), jnp.float32),
    )(be, bv, xs, gate_up_proj, gub3d, down_proj, dpb3d, ws3d)

    out = _make_combine(n_tok)(rows, pos)
    return out.reshape(b, s, h)


# speedup vs baseline: 1.0122x; 1.0122x over previous
import functools

import jax
import jax.numpy as jnp
from jax import lax
from jax.experimental import pallas as pl
from jax.experimental.pallas import tpu as pltpu
from jax.experimental.pallas import tpu_sc as plsc

NC = 2
NS = 16
NW = NC * NS
L = 16

EXP = 8
TOPK = 2
HID = 1024
INNER = 2048
ALPHA = 1.702
LIMIT = 7.0

T = 256
NBLK_MAX = 24
NMETA = 32


def _router_body(x_ref, rw_ref, rb_ref, idx_ref, w_ref):
    x = x_ref[...]
    logits = lax.dot_general(x, rw_ref[...], (((1,), (1,)), ((), ())),
                             preferred_element_type=jnp.float32)
    logits = logits + rb_ref[...][None, :]
    iota = lax.broadcasted_iota(jnp.int32, logits.shape, 1)
    m1 = jnp.max(logits, axis=-1, keepdims=True)
    i1 = jnp.min(jnp.where(logits == m1, iota, EXP), axis=-1, keepdims=True)
    neg = jnp.where(iota == i1, -jnp.inf, logits)
    m2 = jnp.max(neg, axis=-1, keepdims=True)
    i2 = jnp.min(jnp.where(neg == m2, iota, EXP), axis=-1, keepdims=True)
    w1 = 1.0 / (1.0 + jnp.exp(m2 - m1))
    idx_ref[...] = jnp.concatenate([i1, i2], axis=1)
    w_ref[...] = jnp.concatenate([w1, 1.0 - w1], axis=1)


def _gemm_body(be_ref, bv_ref, xs_ref, gu_ref, gub_ref, dp_ref, dpb_ref,
               ws_ref, out_ref):
    b = pl.program_id(0)

    @pl.when(bv_ref[b] == 1)
    def _():
        x = xs_ref[...]
        h1 = lax.dot_general(x, gu_ref[0], (((1,), (1,)), ((), ())),
                             preferred_element_type=jnp.float32)
        h1 = h1 + gub_ref[0]
        gate = jnp.minimum(h1[:, :INNER], LIMIT)
        up = jnp.clip(h1[:, INNER:], -LIMIT, LIMIT)
        glu = gate * (1.0 / (1.0 + jnp.exp(-ALPHA * gate)))
        act = (up + 1.0) * glu
        h2 = lax.dot_general(act, dp_ref[0], (((1,), (1,)), ((), ())),
                             preferred_element_type=jnp.float32)
        h2 = h2 + dpb_ref[0]
        out_ref[...] = h2 * ws_ref[0, 0][:, None]


def _make_dispatch(n_tok):
    n_assign = n_tok * TOPK
    padded = NBLK_MAX * T
    chunk = n_assign // NW
    tchunk = n_tok // NW
    mesh = plsc.VectorSubcoreMesh(core_axis_name="c", subcore_axis_name="s")

    @functools.partial(
        pl.kernel, mesh=mesh,
        compiler_params=pltpu.CompilerParams(needs_layout_passes=False),
        out_type=(jax.ShapeDtypeStruct((n_assign,), jnp.int32),
                  jax.ShapeDtypeStruct((padded, HID), jnp.float32),
                  jax.ShapeDtypeStruct((padded,), jnp.float32),
                  jax.ShapeDtypeStruct((NMETA,), jnp.int32),
                  jax.ShapeDtypeStruct((NMETA,), jnp.int32)),
        scratch_types=[
            pltpu.VMEM((n_assign,), jnp.int32),
            pltpu.VMEM((chunk,), jnp.float32),
            pltpu.VMEM((L,), jnp.int32),
            pltpu.VMEM((L,), jnp.int32),
            pltpu.VMEM((chunk,), jnp.int32),
            pltpu.VMEM((tchunk,), jnp.int32),
            pltpu.VMEM((tchunk,), jnp.int32),
            pltpu.VMEM((tchunk, HID), jnp.float32),
            pltpu.VMEM((2 * L,), jnp.int32),
            pltpu.VMEM((NMETA,), jnp.int32),
            pltpu.VMEM((NMETA,), jnp.int32),
            pltpu.SemaphoreType.DMA,
            pltpu.SemaphoreType.DMA,
        ])
    def dispatch(x_hbm, eid_hbm, wflat_hbm, pos_hbm, xs_hbm, wsort_hbm,
                 be_hbm, bv_hbm, eid_v, w_v, cnt_v, tab_v, slot_v, ev_v,
                 od_v, rows_v, end_v, meta_v, metb_v, sem, sem2):
        wid = lax.axis_index("s") * NC + lax.axis_index("c")
        base = wid * chunk
        rows_cp = pltpu.async_copy(
            x_hbm.at[pl.ds(wid * tchunk, tchunk)], rows_v, sem2)
        pltpu.sync_copy(eid_hbm, eid_v)
        pltpu.sync_copy(wflat_hbm.at[pl.ds(base, chunk)], w_v)
        ones = jnp.ones((L,), jnp.int32)
        cnt_v[...] = jnp.zeros((L,), jnp.int32)
        npre = wid * (chunk // L)

        def count_step(v, carry):
            vec = eid_v[pl.ds(v * L, L)]
            plsc.addupdate_scatter(cnt_v, [vec], ones)
            return carry

        lax.fori_loop(0, npre, count_step, 0, unroll=False)
        for j in range(chunk // L):
            vec = eid_v[pl.ds(base + j * L, L)]
            g = plsc.load_gather(cnt_v, [vec])
            wv = jnp.zeros((L,), jnp.int32)
            for e in range(EXP):
                m = vec == e
                c = jnp.cumsum(m.astype(jnp.int32))
                wv = jnp.where(m, c - 1, wv)
            slot_v[pl.ds(j * L, L)] = g + wv
            plsc.addupdate_scatter(cnt_v, [vec], ones)
        lax.fori_loop(npre + chunk // L, n_assign // L, count_step, 0,
                      unroll=False)
        total = cnt_v[...]
        nblk = (total + (T - 1)) // T
        end_incl = jnp.cumsum(nblk)
        tab_v[...] = (end_incl - nblk) * T
        lane = jnp.arange(L, dtype=jnp.int32)
        for j in range(chunk // L):
            vec = eid_v[pl.ds(base + j * L, L)]
            off = plsc.load_gather(tab_v, [vec])
            slot = slot_v[pl.ds(j * L, L)] + off
            slot_v[pl.ds(j * L, L)] = slot
            dst = j * (L // 2) + (lane >> 1)
            plsc.store_scatter(ev_v, [dst], slot, mask=(lane & 1) == 0)
            plsc.store_scatter(od_v, [dst], slot, mask=(lane & 1) == 1)
        pltpu.sync_copy(slot_v, pos_hbm.at[pl.ds(base, chunk)])
        rows_cp.wait()
        cp_e = pltpu.async_copy(rows_v, xs_hbm.at[ev_v], sem)
        cp_o = pltpu.async_copy(rows_v, xs_hbm.at[od_v], sem2)
        pltpu.async_copy(w_v, wsort_hbm.at[slot_v], sem).wait()
        cp_e.wait()
        cp_o.wait()

        @pl.when(wid == 0)
        def _():
            end_v[pl.ds(L, L)] = end_incl
            sel7 = jnp.full((L,), L + EXP - 1, jnp.int32)
            tot_blk = plsc.load_gather(end_v, [sel7])
            for c in range(NMETA // L):
                bvec = jnp.arange(L, dtype=jnp.int32) + c * L
                beff = jnp.minimum(bvec, tot_blk - 1)
                cv = jnp.zeros((L,), jnp.int32)
                for e in range(EXP):
                    end_e = plsc.load_gather(
                        end_v, [jnp.full((L,), L + e, jnp.int32)])
                    cv = cv + (beff >= end_e).astype(jnp.int32)
                meta_v[pl.ds(c * L, L)] = jnp.minimum(cv, EXP - 1)
                metb_v[pl.ds(c * L, L)] = (bvec < tot_blk).astype(jnp.int32)
            pltpu.sync_copy(meta_v, be_hbm)
            pltpu.sync_copy(metb_v, bv_hbm)

    return dispatch


def _make_combine(n_tok):
    padded = NBLK_MAX * T
    tchunk = n_tok // NW
    tc = 16
    mesh = plsc.VectorSubcoreMesh(core_axis_name="c", subcore_axis_name="s")

    @functools.partial(
        pl.kernel, mesh=mesh,
        out_type=jax.ShapeDtypeStruct((n_tok, HID), jnp.float32),
        scratch_types=[
            pltpu.VMEM((2 * tchunk,), jnp.int32),
            pltpu.VMEM((2 * tc, HID), jnp.float32),
            pltpu.VMEM((tc, HID), jnp.float32),
            pltpu.SemaphoreType.DMA,
        ])
    def combine(rows_hbm, pos_hbm, out_hbm, idx_v, pr_v, o_v, sem):
        wid = lax.axis_index("s") * NC + lax.axis_index("c")
        baset = wid * tchunk
        pltpu.sync_copy(pos_hbm.at[pl.ds(baset * TOPK, TOPK * tchunk)], idx_v)
        for c in range(tchunk // tc):
            pltpu.async_copy(
                rows_hbm.at[idx_v.at[pl.ds(c * TOPK * tc, TOPK * tc)]],
                pr_v, sem).wait()

            def tok_step(t, carry):
                for g in range(HID // L):
                    o_v[t, pl.ds(g * L, L)] = (
                        pr_v[2 * t, pl.ds(g * L, L)]
                        + pr_v[2 * t + 1, pl.ds(g * L, L)])
                return carry

            lax.fori_loop(0, tc, tok_step, 0, unroll=False)
            pltpu.sync_copy(o_v, out_hbm.at[pl.ds(baset + c * tc, tc)])

    return combine


def kernel(hidden_states, router_weight, router_bias, gate_up_proj,
           gate_up_proj_bias, down_proj, down_proj_bias):
    b, s, h = hidden_states.shape
    n_tok = b * s
    n_assign = n_tok * TOPK
    padded = NBLK_MAX * T
    x = hidden_states.reshape(n_tok, h)

    idx, w = pl.pallas_call(
        _router_body,
        out_shape=(jax.ShapeDtypeStruct((n_tok, TOPK), jnp.int32),
                   jax.ShapeDtypeStruct((n_tok, TOPK), jnp.float32)),
    )(x, router_weight, router_bias)

    eid = idx.reshape(n_assign)
    wflat = w.reshape(n_assign)

    pos, xs, wsort, be, bv = _make_dispatch(n_tok)(x, eid, wflat)

    ws3d = wsort.reshape(NBLK_MAX, 1, T)
    gub3d = gate_up_proj_bias.reshape(EXP, 1, 2 * INNER)
    dpb3d = down_proj_bias.reshape(EXP, 1, HID)
    grid_spec = pltpu.PrefetchScalarGridSpec(
        num_scalar_prefetch=2,
        grid=(NBLK_MAX,),
        in_specs=[
            pl.BlockSpec((T, HID), lambda i, be, bv: (i, 0)),
            pl.BlockSpec((1, 2 * INNER, HID), lambda i, be, bv: (be[i], 0, 0)),
            pl.BlockSpec((1, 1, 2 * INNER), lambda i, be, bv: (be[i], 0, 0)),
            pl.BlockSpec((1, HID, INNER), lambda i, be, bv: (be[i], 0, 0)),
            pl.BlockSpec((1, 1, HID), lambda i, be, bv: (be[i], 0, 0)),
            pl.BlockSpec((1, 1, T), lambda i, be, bv: (i, 0, 0)),
        ],
        out_specs=pl.BlockSpec((T, HID), lambda i, be, bv: (i, 0)),
    )
    rows = pl.pallas_call(
        _gemm_body,
        grid_spec=grid_spec,
        out_shape=jax.ShapeDtypeStruct((padded, HID), jnp.float32),
    )(be, bv, xs, gate_up_proj, gub3d, down_proj, dpb3d, ws3d)

    out = _make_combine(n_tok)(rows, pos)
    return out.reshape(b, s, h)


# speedup vs baseline: 1.0510x; 1.0384x over previous
import functools

import jax
import jax.numpy as jnp
from jax import lax
from jax.experimental import pallas as pl
from jax.experimental.pallas import tpu as pltpu
from jax.experimental.pallas import tpu_sc as plsc

NC = 2
NS = 16
NW = NC * NS
L = 16

EXP = 8
TOPK = 2
HID = 1024
INNER = 2048
ALPHA = 1.702
LIMIT = 7.0

T = 256
NBLK_MAX = 24
NMETA = 32


def _router_body(x_ref, rw_ref, rb_ref, idx_ref, w_ref):
    x = x_ref[...]
    logits = lax.dot_general(x, rw_ref[...], (((1,), (1,)), ((), ())),
                             preferred_element_type=jnp.float32)
    logits = logits + rb_ref[...][None, :]
    iota = lax.broadcasted_iota(jnp.int32, logits.shape, 1)
    m1 = jnp.max(logits, axis=-1, keepdims=True)
    i1 = jnp.min(jnp.where(logits == m1, iota, EXP), axis=-1, keepdims=True)
    neg = jnp.where(iota == i1, -jnp.inf, logits)
    m2 = jnp.max(neg, axis=-1, keepdims=True)
    i2 = jnp.min(jnp.where(neg == m2, iota, EXP), axis=-1, keepdims=True)
    w1 = 1.0 / (1.0 + jnp.exp(m2 - m1))
    idx_ref[...] = jnp.concatenate([i1, i2], axis=1)
    w_ref[...] = jnp.concatenate([w1, 1.0 - w1], axis=1)


def _gemm_body(be_ref, bv_ref, xs_ref, gu_ref, gub_ref, dp_ref, dpb_ref,
               ws_ref, out_ref):
    b = pl.program_id(0)

    @pl.when(bv_ref[b] == 1)
    def _():
        x = xs_ref[...]
        h1 = lax.dot_general(x, gu_ref[0], (((1,), (1,)), ((), ())),
                             preferred_element_type=jnp.float32)
        h1 = h1 + gub_ref[0]
        gate = jnp.minimum(h1[:, :INNER], LIMIT)
        up = jnp.clip(h1[:, INNER:], -LIMIT, LIMIT)
        glu = gate * (1.0 / (1.0 + jnp.exp(-ALPHA * gate)))
        act = (up + 1.0) * glu
        h2 = lax.dot_general(act, dp_ref[0], (((1,), (1,)), ((), ())),
                             preferred_element_type=jnp.float32)
        h2 = h2 + dpb_ref[0]
        out_ref[...] = h2 * ws_ref[0, 0][:, None]


def _make_dispatch(n_tok):
    n_assign = n_tok * TOPK
    padded = NBLK_MAX * T
    chunk = n_assign // NW
    tchunk = n_tok // NW
    mesh = plsc.VectorSubcoreMesh(core_axis_name="c", subcore_axis_name="s")

    @functools.partial(
        pl.kernel, mesh=mesh,
        compiler_params=pltpu.CompilerParams(needs_layout_passes=False),
        out_type=(jax.ShapeDtypeStruct((n_assign,), jnp.int32),
                  jax.ShapeDtypeStruct((padded, HID), jnp.float32),
                  jax.ShapeDtypeStruct((padded,), jnp.float32),
                  jax.ShapeDtypeStruct((NMETA,), jnp.int32),
                  jax.ShapeDtypeStruct((NMETA,), jnp.int32)),
        scratch_types=[
            pltpu.VMEM((n_assign,), jnp.int32),
            pltpu.VMEM((chunk,), jnp.float32),
            pltpu.VMEM((L,), jnp.int32),
            pltpu.VMEM((L,), jnp.int32),
            pltpu.VMEM((chunk,), jnp.int32),
            pltpu.VMEM((tchunk,), jnp.int32),
            pltpu.VMEM((tchunk,), jnp.int32),
            pltpu.VMEM((tchunk, HID), jnp.float32),
            pltpu.VMEM((2 * L,), jnp.int32),
            pltpu.VMEM((NMETA,), jnp.int32),
            pltpu.VMEM((NMETA,), jnp.int32),
            pltpu.SemaphoreType.DMA,
            pltpu.SemaphoreType.DMA,
        ])
    def dispatch(x_hbm, eid_hbm, wflat_hbm, pos_hbm, xs_hbm, wsort_hbm,
                 be_hbm, bv_hbm, eid_v, w_v, cnt_v, tab_v, slot_v, ev_v,
                 od_v, rows_v, end_v, meta_v, metb_v, sem, sem2):
        wid = lax.axis_index("s") * NC + lax.axis_index("c")
        base = wid * chunk
        rows_cp = pltpu.async_copy(
            x_hbm.at[pl.ds(wid * tchunk, tchunk)], rows_v, sem2)
        pltpu.sync_copy(eid_hbm, eid_v)
        pltpu.sync_copy(wflat_hbm.at[pl.ds(base, chunk)], w_v)
        ones = jnp.ones((L,), jnp.int32)
        cnt_v[...] = jnp.zeros((L,), jnp.int32)
        npre = wid * (chunk // L)

        def count_step(v, carry):
            vec = eid_v[pl.ds(v * L, L)]
            plsc.addupdate_scatter(cnt_v, [vec], ones)
            return carry

        lax.fori_loop(0, npre, count_step, 0, unroll=False)
        for j in range(chunk // L):
            vec = eid_v[pl.ds(base + j * L, L)]
            g = plsc.load_gather(cnt_v, [vec])
            wv = jnp.zeros((L,), jnp.int32)
            for e in range(EXP):
                m = vec == e
                c = jnp.cumsum(m.astype(jnp.int32))
                wv = jnp.where(m, c - 1, wv)
            slot_v[pl.ds(j * L, L)] = g + wv
            plsc.addupdate_scatter(cnt_v, [vec], ones)
        lax.fori_loop(npre + chunk // L, n_assign // L, count_step, 0,
                      unroll=False)
        total = cnt_v[...]
        nblk = (total + (T - 1)) // T
        end_incl = jnp.cumsum(nblk)
        tab_v[...] = (end_incl - nblk) * T
        lane = jnp.arange(L, dtype=jnp.int32)
        for j in range(chunk // L):
            vec = eid_v[pl.ds(base + j * L, L)]
            off = plsc.load_gather(tab_v, [vec])
            slot = slot_v[pl.ds(j * L, L)] + off
            slot_v[pl.ds(j * L, L)] = slot
            dst = j * (L // 2) + (lane >> 1)
            plsc.store_scatter(ev_v, [dst], slot, mask=(lane & 1) == 0)
            plsc.store_scatter(od_v, [dst], slot, mask=(lane & 1) == 1)
        pltpu.sync_copy(slot_v, pos_hbm.at[pl.ds(base, chunk)])
        rows_cp.wait()
        cp_e = pltpu.async_copy(rows_v, xs_hbm.at[ev_v], sem)
        cp_o = pltpu.async_copy(rows_v, xs_hbm.at[od_v], sem2)
        pltpu.async_copy(w_v, wsort_hbm.at[slot_v], sem).wait()
        cp_e.wait()
        cp_o.wait()

        @pl.when(wid == 0)
        def _():
            end_v[pl.ds(L, L)] = end_incl
            sel7 = jnp.full((L,), L + EXP - 1, jnp.int32)
            tot_blk = plsc.load_gather(end_v, [sel7])
            for c in range(NMETA // L):
                bvec = jnp.arange(L, dtype=jnp.int32) + c * L
                beff = jnp.minimum(bvec, tot_blk - 1)
                cv = jnp.zeros((L,), jnp.int32)
                for e in range(EXP):
                    end_e = plsc.load_gather(
                        end_v, [jnp.full((L,), L + e, jnp.int32)])
                    cv = cv + (beff >= end_e).astype(jnp.int32)
                meta_v[pl.ds(c * L, L)] = jnp.minimum(cv, EXP - 1)
                metb_v[pl.ds(c * L, L)] = (bvec < tot_blk).astype(jnp.int32)
            pltpu.sync_copy(meta_v, be_hbm)
            pltpu.sync_copy(metb_v, bv_hbm)

    return dispatch


def _make_combine(n_tok):
    padded = NBLK_MAX * T
    tchunk = n_tok // NW
    tc = 16
    mesh = plsc.VectorSubcoreMesh(core_axis_name="c", subcore_axis_name="s")

    nchunk = tchunk // tc

    @functools.partial(
        pl.kernel, mesh=mesh,
        out_type=jax.ShapeDtypeStruct((n_tok, HID), jnp.float32),
        scratch_types=[
            pltpu.VMEM((2 * tchunk,), jnp.int32),
            pltpu.VMEM((2 * tc, HID), jnp.float32),
            pltpu.VMEM((2 * tc, HID), jnp.float32),
            pltpu.VMEM((tc, HID), jnp.float32),
            pltpu.VMEM((tc, HID), jnp.float32),
            pltpu.SemaphoreType.DMA,
            pltpu.SemaphoreType.DMA,
            pltpu.SemaphoreType.DMA,
            pltpu.SemaphoreType.DMA,
        ])
    def combine(rows_hbm, pos_hbm, out_hbm, idx_v, pr0_v, pr1_v, o0_v, o1_v,
                sem0, sem1, osem0, osem1):
        wid = lax.axis_index("s") * NC + lax.axis_index("c")
        baset = wid * tchunk
        pltpu.sync_copy(pos_hbm.at[pl.ds(baset * TOPK, TOPK * tchunk)], idx_v)
        prs = [pr0_v, pr1_v]
        outs = [o0_v, o1_v]
        isems = [sem0, sem1]
        osems = [osem0, osem1]

        def start_gather(c):
            return pltpu.async_copy(
                rows_hbm.at[idx_v.at[pl.ds(c * TOPK * tc, TOPK * tc)]],
                prs[c % 2], isems[c % 2])

        gcp = [None] * nchunk
        ocp = [None] * nchunk
        gcp[0] = start_gather(0)
        for c in range(nchunk):
            gcp[c].wait()
            if c + 1 < nchunk:
                gcp[c + 1] = start_gather(c + 1)
            if c >= 2:
                ocp[c - 2].wait()
            pr_v = prs[c % 2]
            o_v = outs[c % 2]

            def tok_step(t, carry):
                for g in range(HID // L):
                    o_v[t, pl.ds(g * L, L)] = (
                        pr_v[2 * t, pl.ds(g * L, L)]
                        + pr_v[2 * t + 1, pl.ds(g * L, L)])
                return carry

            lax.fori_loop(0, tc, tok_step, 0, unroll=False)
            ocp[c] = pltpu.async_copy(
                o_v, out_hbm.at[pl.ds(baset + c * tc, tc)], osems[c % 2])
        ocp[nchunk - 2].wait()
        ocp[nchunk - 1].wait()

    return combine


def kernel(hidden_states, router_weight, router_bias, gate_up_proj,
           gate_up_proj_bias, down_proj, down_proj_bias):
    b, s, h = hidden_states.shape
    n_tok = b * s
    n_assign = n_tok * TOPK
    padded = NBLK_MAX * T
    x = hidden_states.reshape(n_tok, h)

    idx, w = pl.pallas_call(
        _router_body,
        out_shape=(jax.ShapeDtypeStruct((n_tok, TOPK), jnp.int32),
                   jax.ShapeDtypeStruct((n_tok, TOPK), jnp.float32)),
    )(x, router_weight, router_bias)

    eid = idx.reshape(n_assign)
    wflat = w.reshape(n_assign)

    pos, xs, wsort, be, bv = _make_dispatch(n_tok)(x, eid, wflat)

    ws3d = wsort.reshape(NBLK_MAX, 1, T)
    gub3d = gate_up_proj_bias.reshape(EXP, 1, 2 * INNER)
    dpb3d = down_proj_bias.reshape(EXP, 1, HID)
    grid_spec = pltpu.PrefetchScalarGridSpec(
        num_scalar_prefetch=2,
        grid=(NBLK_MAX,),
        in_specs=[
            pl.BlockSpec((T, HID), lambda i, be, bv: (i, 0)),
            pl.BlockSpec((1, 2 * INNER, HID), lambda i, be, bv: (be[i], 0, 0)),
            pl.BlockSpec((1, 1, 2 * INNER), lambda i, be, bv: (be[i], 0, 0)),
            pl.BlockSpec((1, HID, INNER), lambda i, be, bv: (be[i], 0, 0)),
            pl.BlockSpec((1, 1, HID), lambda i, be, bv: (be[i], 0, 0)),
            pl.BlockSpec((1, 1, T), lambda i, be, bv: (i, 0, 0)),
        ],
        out_specs=pl.BlockSpec((T, HID), lambda i, be, bv: (i, 0)),
    )
    rows = pl.pallas_call(
        _gemm_body,
        grid_spec=grid_spec,
        out_shape=jax.ShapeDtypeStruct((padded, HID), jnp.float32),
    )(be, bv, xs, gate_up_proj, gub3d, down_proj, dpb3d, ws3d)

    out = _make_combine(n_tok)(rows, pos)
    return out.reshape(b, s, h)
